# baseline (speedup 1.0000x reference)
import jax
import jax.numpy as jnp
from jax.experimental import pallas as pl
from jax.experimental.pallas import tpu as pltpu

B = 256
N_USER = 50000
D = 128
HID = 128
NCLS = 4


def _sage(src_feat, dst_feat, src_idx, dst_idx, n_dst, pool_w, pool_b, neigh_w, self_w, bias):
    h = jax.nn.relu(src_feat @ pool_w.T + pool_b)
    msg = jnp.take(h, src_idx, axis=0)
    agg = jax.ops.segment_max(msg, dst_idx, num_segments=n_dst)
    agg = jnp.where(jnp.isneginf(agg), 0.0, agg)
    return dst_feat @ self_w.T + agg @ neigh_w.T + bias


def _head_kernel(hg_in_ref, news_feat_ref, lin1_w_ref, lin1_b_ref,
                 lin2_w_ref, lin2_b_ref, cls_w_ref, cls_b_ref, out_ref):
    hg = jnp.maximum(
        jnp.dot(hg_in_ref[...], lin1_w_ref[...].T, preferred_element_type=jnp.float32)
        + lin1_b_ref[...], 0.0)
    news = jnp.maximum(
        jnp.dot(news_feat_ref[...], lin2_w_ref[...].T, preferred_element_type=jnp.float32)
        + lin2_b_ref[...], 0.0)
    z = jnp.concatenate([hg, news], axis=1)
    logits = jnp.dot(z, cls_w_ref[...].T, preferred_element_type=jnp.float32) + cls_b_ref[...]
    m = jnp.max(logits, axis=-1, keepdims=True)
    s = logits - m
    lse = jnp.log(jnp.sum(jnp.exp(s), axis=-1, keepdims=True))
    out_ref[...] = s - lse


def kernel(news_feat, user_feat, p1_pool_w, p1_pool_b, p1_neigh_w, p1_self_w, p1_bias, f1_pool_w, f1_pool_b, f1_neigh_w, f1_self_w, f1_bias, p2_pool_w, p2_pool_b, p2_neigh_w, p2_self_w, p2_bias, f2_pool_w, f2_pool_b, f2_neigh_w, f2_self_w, f2_bias, lin1_w, lin1_b, lin2_w, lin2_b, cls_w, cls_b, posts_src, posts_dst, follows_src, follows_dst, user_graph_ids):
    h_news = jax.nn.relu(_sage(user_feat, news_feat, posts_src, posts_dst, B,
                               p1_pool_w, p1_pool_b, p1_neigh_w, p1_self_w, p1_bias))
    h_user = jax.nn.relu(_sage(user_feat, user_feat, follows_src, follows_dst, N_USER,
                               f1_pool_w, f1_pool_b, f1_neigh_w, f1_self_w, f1_bias))
    h_news2 = _sage(h_user, h_news, posts_src, posts_dst, B,
                    p2_pool_w, p2_pool_b, p2_neigh_w, p2_self_w, p2_bias)
    h_user2 = _sage(h_user, h_user, follows_src, follows_dst, N_USER,
                    f2_pool_w, f2_pool_b, f2_neigh_w, f2_self_w, f2_bias)
    ones = jnp.ones((N_USER,), dtype=jnp.float32)
    cnt = jax.ops.segment_sum(ones, user_graph_ids, num_segments=B)
    hg_user = jax.ops.segment_sum(h_user2, user_graph_ids, num_segments=B) / jnp.maximum(cnt, 1.0)[:, None]
    hg_in = h_news2 + hg_user

    out = pl.pallas_call(
        _head_kernel,
        out_shape=jax.ShapeDtypeStruct((B, NCLS), jnp.float32),
    )(hg_in, news_feat, lin1_w, lin1_b, lin2_w, lin2_b, cls_w, cls_b)
    return out


# speedup vs baseline: 1.6910x; 1.6910x over previous
import functools

import jax
import jax.numpy as jnp
from jax import lax
from jax.experimental import pallas as pl
from jax.experimental.pallas import tpu as pltpu
from jax.experimental.pallas import tpu_sc as plsc

B = 256
N_USER = 50000
D = 128
HID = 128
NCLS = 4
E_POSTS = 100000
E_FOLLOWS = 500000

NW = 32
F_RNG = 800
F_NBINS = 63
F_CAP = 16384
SCAN_CHUNK = 10000
GC = 96
E_PP = 3128
E_POSTS_PAD = E_PP * NW

_mesh = plsc.VectorSubcoreMesh(core_axis_name="c", subcore_axis_name="s")


def _worker_id():
    return lax.axis_index("s") * 2 + lax.axis_index("c")


def _fprep_body(src_hbm, dst_hbm, slab_s, slab_d, counts_hbm,
                src_c, dst_c, sbuf, dbuf, cbuf):
    wid = _worker_id()

    def zero16(i, _):
        z = jnp.zeros((16,), jnp.int32)
        sbuf[pl.ds(i * 16, 16)] = z
        dbuf[pl.ds(i * 16, 16)] = z
        return 0

    lax.fori_loop(0, F_CAP // 16, zero16, 0)

    for k in range(2):
        t = wid + NW * k

        @pl.when(t < F_NBINS)
        def _():
            lo = t * F_RNG
            hi = lo + F_RNG

            def scan_chunk(g, cur):
                pltpu.sync_copy(src_hbm.at[pl.ds(g * SCAN_CHUNK, SCAN_CHUNK)], src_c)
                pltpu.sync_copy(dst_hbm.at[pl.ds(g * SCAN_CHUNK, SCAN_CHUNK)], dst_c)

                def inner(i, cur):
                    sv = src_c[pl.ds(i * 16, 16)]
                    dv = dst_c[pl.ds(i * 16, 16)]
                    m = (dv >= lo) & (dv < hi)
                    mi = jnp.where(m, 1, 0).astype(jnp.int32)
                    cs = plsc.cumsum(mi)
                    pos = cur + cs - mi
                    plsc.store_scatter(sbuf, [pos], sv, mask=m)
                    plsc.store_scatter(dbuf, [pos], dv - lo, mask=m)
                    return jnp.minimum(cur + cs[15], F_CAP - 16)

                return lax.fori_loop(0, SCAN_CHUNK // 16, inner, cur)

            cnt = lax.fori_loop(0, E_FOLLOWS // SCAN_CHUNK, scan_chunk, jnp.int32(0))
            cbuf[pl.ds(0, 16)] = jnp.full((16,), cnt, jnp.int32)
            pltpu.sync_copy(sbuf, slab_s.at[pl.ds(t * F_CAP, F_CAP)])
            pltpu.sync_copy(dbuf, slab_d.at[pl.ds(t * F_CAP, F_CAP)])
            pltpu.sync_copy(cbuf, counts_hbm.at[pl.ds(t * 16, 16)])


_follows_prep = functools.partial(
    pl.kernel,
    out_type=[
        jax.ShapeDtypeStruct((F_NBINS * F_CAP,), jnp.int32),
        jax.ShapeDtypeStruct((F_NBINS * F_CAP,), jnp.int32),
        jax.ShapeDtypeStruct((F_NBINS * 16,), jnp.int32),
    ],
    mesh=_mesh,
    compiler_params=pltpu.CompilerParams(needs_layout_passes=False),
    scratch_types=[
        pltpu.VMEM((SCAN_CHUNK,), jnp.int32),
        pltpu.VMEM((SCAN_CHUNK,), jnp.int32),
        pltpu.VMEM((F_CAP,), jnp.int32),
        pltpu.VMEM((F_CAP,), jnp.int32),
        pltpu.VMEM((16,), jnp.int32),
    ],
)(_fprep_body)


def _gather_rmw(h_hbm, isrc_hbm, idst_hbm, base, cnt, max_off, acc, trash,
                idxS, idxD, rows, semI, semG):
    nch = jnp.maximum((cnt + GC - 1) // GC, 1)
    nch2 = nch + (nch & 1)

    def off(g):
        return jnp.minimum(base + g * GC, max_off)

    def issue_idx(g, b):
        pltpu.async_copy(isrc_hbm.at[pl.ds(off(g), GC)], idxS[b], semI[b])
        pltpu.async_copy(idst_hbm.at[pl.ds(off(g), GC)], idxD[b], semI[b])

    def wait_idx(b):
        pltpu.make_async_copy(isrc_hbm.at[pl.ds(0, GC)], idxS[b], semI[b]).wait()
        pltpu.make_async_copy(idst_hbm.at[pl.ds(0, GC)], idxD[b], semI[b]).wait()

    def issue_gather(b):
        pltpu.async_copy(h_hbm.at[idxS[b]], rows[b], semG[b])

    def wait_gather(b):
        pltpu.make_async_copy(h_hbm.at[idxS[b]], rows[b], semG[b]).wait()

    def process(g, b):
        ne = jnp.clip(cnt - g * GC, 0, GC)

        def chunk16(q, _):
            vd = idxD[b][pl.ds(16 * q, 16)]
            for i in range(16):
                e = 16 * q + i
                d = jnp.where(e < ne, vd[i], trash)
                for f in range(8):
                    sl = pl.ds(f * 16, 16)
                    acc[d, sl] = jnp.maximum(acc[d, sl], rows[b][e, sl])
            return 0

        lax.fori_loop(0, GC // 16, chunk16, 0)

    issue_idx(0, 0)
    issue_idx(1, 1)
    wait_idx(0)
    issue_gather(0)

    def pair(p, _):
        g = 2 * p
        wait_gather(0)
        wait_idx(1)
        issue_gather(1)
        process(g, 0)
        issue_idx(g + 2, 0)
        wait_gather(1)
        wait_idx(0)
        issue_gather(0)
        process(g + 1, 1)
        issue_idx(g + 3, 1)
        return 0

    lax.fori_loop(0, nch2 // 2, pair, 0)
    wait_gather(0)
    wait_idx(1)


def _zero_acc(acc, nrows):
    def zr(i, _):
        z = jnp.zeros((16,), jnp.float32)
        for f in range(8):
            acc[i, pl.ds(f * 16, 16)] = z
        return 0

    lax.fori_loop(0, nrows, zr, 0)


def _fagg_body(h_hbm, slab_s, slab_d, counts_hbm, out_hbm,
               idxS0, idxS1, idxD0, idxD1, rows0, rows1, acc, cbuf,
               semI0, semI1, semG0, semG1):
    wid = _worker_id()
    idxS = (idxS0, idxS1)
    idxD = (idxD0, idxD1)
    rows = (rows0, rows1)
    semI = (semI0, semI1)
    semG = (semG0, semG1)
    tail = N_USER - (F_NBINS - 1) * F_RNG

    for k in range(2):
        t = wid + NW * k

        @pl.when(t < F_NBINS)
        def _():
            _zero_acc(acc, F_RNG + 8)
            pltpu.sync_copy(counts_hbm.at[pl.ds(t * 16, 16)], cbuf)
            cnt = cbuf[pl.ds(0, 16)][0]
            _gather_rmw(h_hbm, slab_s, slab_d, t * F_CAP, cnt,
                        F_NBINS * F_CAP - GC, acc, F_RNG, idxS, idxD, rows, semI, semG)

            @pl.when(t < F_NBINS - 1)
            def _():
                pltpu.sync_copy(acc.at[pl.ds(0, F_RNG)], out_hbm.at[pl.ds(t * F_RNG, F_RNG)])

            @pl.when(t == F_NBINS - 1)
            def _():
                pltpu.sync_copy(acc.at[pl.ds(0, tail)],
                                out_hbm.at[pl.ds((F_NBINS - 1) * F_RNG, tail)])


_follows_agg = functools.partial(
    pl.kernel,
    out_type=jax.ShapeDtypeStruct((N_USER, HID), jnp.float32),
    mesh=_mesh,
    scratch_types=[
        pltpu.VMEM((GC,), jnp.int32),
        pltpu.VMEM((GC,), jnp.int32),
        pltpu.VMEM((GC,), jnp.int32),
        pltpu.VMEM((GC,), jnp.int32),
        pltpu.VMEM((GC, HID), jnp.float32),
        pltpu.VMEM((GC, HID), jnp.float32),
        pltpu.VMEM((F_RNG + 8, HID), jnp.float32),
        pltpu.VMEM((16,), jnp.int32),
        pltpu.SemaphoreType.DMA,
        pltpu.SemaphoreType.DMA,
        pltpu.SemaphoreType.DMA,
        pltpu.SemaphoreType.DMA,
    ],
)(_fagg_body)


def _pagg_body(h_hbm, psrc_hbm, pdst_hbm, part_hbm,
               idxS0, idxS1, idxD0, idxD1, rows0, rows1, acc,
               semI0, semI1, semG0, semG1):
    wid = _worker_id()
    idxS = (idxS0, idxS1)
    idxD = (idxD0, idxD1)
    rows = (rows0, rows1)
    semI = (semI0, semI1)
    semG = (semG0, semG1)

    _zero_acc(acc, B + 8)
    _gather_rmw(h_hbm, psrc_hbm, pdst_hbm, wid * E_PP, jnp.int32(E_PP),
                E_POSTS_PAD - GC, acc, B, idxS, idxD, rows, semI, semG)
    pltpu.sync_copy(acc.at[pl.ds(0, B)], part_hbm.at[wid])


_posts_agg = functools.partial(
    pl.kernel,
    out_type=jax.ShapeDtypeStruct((NW, B, HID), jnp.float32),
    mesh=_mesh,
    scratch_types=[
        pltpu.VMEM((GC,), jnp.int32),
        pltpu.VMEM((GC,), jnp.int32),
        pltpu.VMEM((GC,), jnp.int32),
        pltpu.VMEM((GC,), jnp.int32),
        pltpu.VMEM((GC, HID), jnp.float32),
        pltpu.VMEM((GC, HID), jnp.float32),
        pltpu.VMEM((B + 8, HID), jnp.float32),
        pltpu.SemaphoreType.DMA,
        pltpu.SemaphoreType.DMA,
        pltpu.SemaphoreType.DMA,
        pltpu.SemaphoreType.DMA,
    ],
)(_pagg_body)


def _head_kernel(hg_in_ref, news_feat_ref, lin1_w_ref, lin1_b_ref,
                 lin2_w_ref, lin2_b_ref, cls_w_ref, cls_b_ref, out_ref):
    hg = jnp.maximum(
        jnp.dot(hg_in_ref[...], lin1_w_ref[...].T, preferred_element_type=jnp.float32)
        + lin1_b_ref[...], 0.0)
    news = jnp.maximum(
        jnp.dot(news_feat_ref[...], lin2_w_ref[...].T, preferred_element_type=jnp.float32)
        + lin2_b_ref[...], 0.0)
    z = jnp.concatenate([hg, news], axis=1)
    logits = jnp.dot(z, cls_w_ref[...].T, preferred_element_type=jnp.float32) + cls_b_ref[...]
    m = jnp.max(logits, axis=-1, keepdims=True)
    s = logits - m
    lse = jnp.log(jnp.sum(jnp.exp(s), axis=-1, keepdims=True))
    out_ref[...] = s - lse


def kernel(news_feat, user_feat, p1_pool_w, p1_pool_b, p1_neigh_w, p1_self_w, p1_bias, f1_pool_w, f1_pool_b, f1_neigh_w, f1_self_w, f1_bias, p2_pool_w, p2_pool_b, p2_neigh_w, p2_self_w, p2_bias, f2_pool_w, f2_pool_b, f2_neigh_w, f2_self_w, f2_bias, lin1_w, lin1_b, lin2_w, lin2_b, cls_w, cls_b, posts_src, posts_dst, follows_src, follows_dst, user_graph_ids):
    relu = jax.nn.relu

    npad = E_POSTS_PAD - E_POSTS
    psrc = jnp.concatenate([posts_src, jnp.zeros((npad,), jnp.int32)])
    pdst = jnp.concatenate([posts_dst, jnp.full((npad,), B, jnp.int32)])

    slab_s, slab_d, counts = _follows_prep(follows_src, follows_dst)

    hp1 = relu(user_feat @ p1_pool_w.T + p1_pool_b)
    hf1 = relu(user_feat @ f1_pool_w.T + f1_pool_b)

    aggp1 = jnp.max(_posts_agg(hp1, psrc, pdst), axis=0)
    aggf1 = _follows_agg(hf1, slab_s, slab_d, counts)

    h_news = relu(news_feat @ p1_self_w.T + aggp1 @ p1_neigh_w.T + p1_bias)
    h_user = relu(user_feat @ f1_self_w.T + aggf1 @ f1_neigh_w.T + f1_bias)

    hp2 = relu(h_user @ p2_pool_w.T + p2_pool_b)
    hf2 = relu(h_user @ f2_pool_w.T + f2_pool_b)

    aggp2 = jnp.max(_posts_agg(hp2, psrc, pdst), axis=0)
    aggf2 = _follows_agg(hf2, slab_s, slab_d, counts)

    h_news2 = h_news @ p2_self_w.T + aggp2 @ p2_neigh_w.T + p2_bias
    h_user2 = h_user @ f2_self_w.T + aggf2 @ f2_neigh_w.T + f2_bias

    ones = jnp.ones((N_USER,), dtype=jnp.float32)
    cnt = jax.ops.segment_sum(ones, user_graph_ids, num_segments=B)
    hg_user = jax.ops.segment_sum(h_user2, user_graph_ids, num_segments=B) / jnp.maximum(cnt, 1.0)[:, None]
    hg_in = h_news2 + hg_user

    out = pl.pallas_call(
        _head_kernel,
        out_shape=jax.ShapeDtypeStruct((B, NCLS), jnp.float32),
    )(hg_in, news_feat, lin1_w, lin1_b, lin2_w, lin2_b, cls_w, cls_b)
    return out


# speedup vs baseline: 1.8450x; 1.0911x over previous
import functools

import jax
import jax.numpy as jnp
from jax import lax
from jax.experimental import pallas as pl
from jax.experimental.pallas import tpu as pltpu
from jax.experimental.pallas import tpu_sc as plsc

B = 256
N_USER = 50000
D = 128
HID = 128
NCLS = 4
E_POSTS = 100000
E_FOLLOWS = 500000

NW = 32
F_RNG = 800
F_NBINS = 63
F_CAP = 16384
SCAN_CHUNK = 10000
GC = 96
E_PP = 3128
E_POSTS_PAD = E_PP * NW

_mesh = plsc.VectorSubcoreMesh(core_axis_name="c", subcore_axis_name="s")


def _worker_id():
    return lax.axis_index("s") * 2 + lax.axis_index("c")


def _fprep_body(src_hbm, dst_hbm, slab_s, slab_d, counts_hbm,
                src_c, dst_c, sbuf, dbuf, cbuf):
    wid = _worker_id()

    def zero16(i, _):
        z = jnp.zeros((16,), jnp.int32)
        sbuf[pl.ds(i * 16, 16)] = z
        dbuf[pl.ds(i * 16, 16)] = z
        return 0

    lax.fori_loop(0, F_CAP // 16, zero16, 0)

    for k in range(2):
        t = wid + NW * k

        @pl.when(t < F_NBINS)
        def _():
            lo = t * F_RNG
            hi = lo + F_RNG

            def scan_chunk(g, cur):
                pltpu.sync_copy(src_hbm.at[pl.ds(g * SCAN_CHUNK, SCAN_CHUNK)], src_c)
                pltpu.sync_copy(dst_hbm.at[pl.ds(g * SCAN_CHUNK, SCAN_CHUNK)], dst_c)

                def inner(i, cur):
                    sv = src_c[pl.ds(i * 16, 16)]
                    dv = dst_c[pl.ds(i * 16, 16)]
                    m = (dv >= lo) & (dv < hi)
                    mi = jnp.where(m, 1, 0).astype(jnp.int32)
                    cs = plsc.cumsum(mi)
                    pos = cur + cs - mi
                    plsc.store_scatter(sbuf, [pos], sv, mask=m)
                    plsc.store_scatter(dbuf, [pos], dv - lo, mask=m)
                    return jnp.minimum(cur + cs[15], F_CAP - 16)

                return lax.fori_loop(0, SCAN_CHUNK // 16, inner, cur, unroll=4)

            cnt = lax.fori_loop(0, E_FOLLOWS // SCAN_CHUNK, scan_chunk, jnp.int32(0))
            cbuf[pl.ds(0, 16)] = jnp.full((16,), cnt, jnp.int32)
            pltpu.sync_copy(sbuf, slab_s.at[pl.ds(t * F_CAP, F_CAP)])
            pltpu.sync_copy(dbuf, slab_d.at[pl.ds(t * F_CAP, F_CAP)])
            pltpu.sync_copy(cbuf, counts_hbm.at[pl.ds(t * 16, 16)])


_follows_prep = functools.partial(
    pl.kernel,
    out_type=[
        jax.ShapeDtypeStruct((F_NBINS * F_CAP,), jnp.int32),
        jax.ShapeDtypeStruct((F_NBINS * F_CAP,), jnp.int32),
        jax.ShapeDtypeStruct((F_NBINS * 16,), jnp.int32),
    ],
    mesh=_mesh,
    compiler_params=pltpu.CompilerParams(needs_layout_passes=False),
    scratch_types=[
        pltpu.VMEM((SCAN_CHUNK,), jnp.int32),
        pltpu.VMEM((SCAN_CHUNK,), jnp.int32),
        pltpu.VMEM((F_CAP,), jnp.int32),
        pltpu.VMEM((F_CAP,), jnp.int32),
        pltpu.VMEM((16,), jnp.int32),
    ],
)(_fprep_body)


def _gather_rmw(h_hbm, isrc_hbm, idst_hbm, base, cnt, max_off, acc, trash,
                idxS, idxD, rows, semI, semG):
    nch = jnp.maximum((cnt + GC - 1) // GC, 1)
    nch2 = nch + (nch & 1)

    def off(g):
        return jnp.minimum(base + g * GC, max_off)

    def issue_idx(g, b):
        pltpu.async_copy(isrc_hbm.at[pl.ds(off(g), GC)], idxS[b], semI[b])
        pltpu.async_copy(idst_hbm.at[pl.ds(off(g), GC)], idxD[b], semI[b])

    def wait_idx(b):
        pltpu.make_async_copy(isrc_hbm.at[pl.ds(0, GC)], idxS[b], semI[b]).wait()
        pltpu.make_async_copy(idst_hbm.at[pl.ds(0, GC)], idxD[b], semI[b]).wait()

    def issue_gather(b):
        pltpu.async_copy(h_hbm.at[idxS[b]], rows[b], semG[b])

    def wait_gather(b):
        pltpu.make_async_copy(h_hbm.at[idxS[b]], rows[b], semG[b]).wait()

    def process(g, b):
        ne = jnp.clip(cnt - g * GC, 0, GC)

        def chunk16(q, _):
            vd = idxD[b][pl.ds(16 * q, 16)]
            for i in range(16):
                e = 16 * q + i
                d = jnp.where(e < ne, vd[i], trash)
                for ff in range(8):
                    f = (ff + i) % 8
                    sl = pl.ds(f * 16, 16)
                    acc[d, sl] = jnp.maximum(acc[d, sl], rows[b][e, sl])
            return 0

        lax.fori_loop(0, GC // 16, chunk16, 0)

    issue_idx(0, 0)
    issue_idx(1, 1)
    wait_idx(0)
    issue_gather(0)

    def pair(p, _):
        g = 2 * p
        wait_gather(0)
        wait_idx(1)
        issue_gather(1)
        process(g, 0)
        issue_idx(g + 2, 0)
        wait_gather(1)
        wait_idx(0)
        issue_gather(0)
        process(g + 1, 1)
        issue_idx(g + 3, 1)
        return 0

    lax.fori_loop(0, nch2 // 2, pair, 0)
    wait_gather(0)
    wait_idx(1)


def _zero_acc(acc, nrows):
    def zr(i, _):
        z = jnp.zeros((16,), jnp.float32)
        for f in range(8):
            acc[i, pl.ds(f * 16, 16)] = z
        return 0

    lax.fori_loop(0, nrows, zr, 0)


def _fagg_body(h_hbm, slab_s, slab_d, counts_hbm, out_hbm,
               idxS0, idxS1, idxD0, idxD1, rows0, rows1, acc, cbuf,
               semI0, semI1, semG0, semG1):
    wid = _worker_id()
    idxS = (idxS0, idxS1)
    idxD = (idxD0, idxD1)
    rows = (rows0, rows1)
    semI = (semI0, semI1)
    semG = (semG0, semG1)
    tail = N_USER - (F_NBINS - 1) * F_RNG

    for k in range(2):
        t = wid + NW * k

        @pl.when(t < F_NBINS)
        def _():
            _zero_acc(acc, F_RNG + 8)
            pltpu.sync_copy(counts_hbm.at[pl.ds(t * 16, 16)], cbuf)
            cnt = cbuf[pl.ds(0, 16)][0]
            _gather_rmw(h_hbm, slab_s, slab_d, t * F_CAP, cnt,
                        F_NBINS * F_CAP - GC, acc, F_RNG, idxS, idxD, rows, semI, semG)

            @pl.when(t < F_NBINS - 1)
            def _():
                pltpu.sync_copy(acc.at[pl.ds(0, F_RNG)], out_hbm.at[pl.ds(t * F_RNG, F_RNG)])

            @pl.when(t == F_NBINS - 1)
            def _():
                pltpu.sync_copy(acc.at[pl.ds(0, tail)],
                                out_hbm.at[pl.ds((F_NBINS - 1) * F_RNG, tail)])


_follows_agg = functools.partial(
    pl.kernel,
    out_type=jax.ShapeDtypeStruct((N_USER, HID), jnp.float32),
    mesh=_mesh,
    scratch_types=[
        pltpu.VMEM((GC,), jnp.int32),
        pltpu.VMEM((GC,), jnp.int32),
        pltpu.VMEM((GC,), jnp.int32),
        pltpu.VMEM((GC,), jnp.int32),
        pltpu.VMEM((GC, HID), jnp.float32),
        pltpu.VMEM((GC, HID), jnp.float32),
        pltpu.VMEM((F_RNG + 8, HID), jnp.float32),
        pltpu.VMEM((16,), jnp.int32),
        pltpu.SemaphoreType.DMA,
        pltpu.SemaphoreType.DMA,
        pltpu.SemaphoreType.DMA,
        pltpu.SemaphoreType.DMA,
    ],
)(_fagg_body)


def _pagg_body(h_hbm, psrc_hbm, pdst_hbm, part_hbm,
               idxS0, idxS1, idxD0, idxD1, rows0, rows1, acc,
               semI0, semI1, semG0, semG1):
    wid = _worker_id()
    idxS = (idxS0, idxS1)
    idxD = (idxD0, idxD1)
    rows = (rows0, rows1)
    semI = (semI0, semI1)
    semG = (semG0, semG1)

    _zero_acc(acc, B + 8)
    _gather_rmw(h_hbm, psrc_hbm, pdst_hbm, wid * E_PP, jnp.int32(E_PP),
                E_POSTS_PAD - GC, acc, B, idxS, idxD, rows, semI, semG)
    pltpu.sync_copy(acc.at[pl.ds(0, B)], part_hbm.at[wid])


_posts_agg = functools.partial(
    pl.kernel,
    out_type=jax.ShapeDtypeStruct((NW, B, HID), jnp.float32),
    mesh=_mesh,
    scratch_types=[
        pltpu.VMEM((GC,), jnp.int32),
        pltpu.VMEM((GC,), jnp.int32),
        pltpu.VMEM((GC,), jnp.int32),
        pltpu.VMEM((GC,), jnp.int32),
        pltpu.VMEM((GC, HID), jnp.float32),
        pltpu.VMEM((GC, HID), jnp.float32),
        pltpu.VMEM((B + 8, HID), jnp.float32),
        pltpu.SemaphoreType.DMA,
        pltpu.SemaphoreType.DMA,
        pltpu.SemaphoreType.DMA,
        pltpu.SemaphoreType.DMA,
    ],
)(_pagg_body)


RB = 2000
NRB = N_USER // RB


def _dot_t(x, w):
    return jax.lax.dot_general(x, w, (((1,), (1,)), ((), ())),
                               preferred_element_type=jnp.float32)


def _mm2_body(x_ref, w1_ref, b1_ref, w2_ref, b2_ref, o1_ref, o2_ref):
    x = x_ref[...]
    o1_ref[...] = jnp.maximum(_dot_t(x, w1_ref[...]) + b1_ref[...], 0.0)
    o2_ref[...] = jnp.maximum(_dot_t(x, w2_ref[...]) + b2_ref[...], 0.0)


def _pool2_tc(x, w1, b1, w2, b2):
    blk = pl.BlockSpec((RB, D), lambda i: (i, 0))
    full = pl.BlockSpec(None, lambda i: (0, 0))
    return pl.pallas_call(
        _mm2_body,
        grid=(NRB,),
        in_specs=[blk, full, full, full, full],
        out_specs=[blk, blk],
        out_shape=[jax.ShapeDtypeStruct((N_USER, HID), jnp.float32)] * 2,
    )(x, w1, b1.reshape(1, -1), w2, b2.reshape(1, -1))


def _huser_body(u_ref, agg_ref, sw_ref, nw_ref, b_ref,
                pw_ref, pb_ref, fw_ref, fb_ref,
                hu_ref, hp_ref, hf_ref):
    hu = jnp.maximum(_dot_t(u_ref[...], sw_ref[...])
                     + _dot_t(agg_ref[...], nw_ref[...]) + b_ref[...], 0.0)
    hu_ref[...] = hu
    hp_ref[...] = jnp.maximum(_dot_t(hu, pw_ref[...]) + pb_ref[...], 0.0)
    hf_ref[...] = jnp.maximum(_dot_t(hu, fw_ref[...]) + fb_ref[...], 0.0)


def _huser_tc(u, agg, sw, nw, b, pw, pb, fw, fb):
    blk = pl.BlockSpec((RB, D), lambda i: (i, 0))
    full = pl.BlockSpec(None, lambda i: (0, 0))
    return pl.pallas_call(
        _huser_body,
        grid=(NRB,),
        in_specs=[blk, blk, full, full, full, full, full, full, full],
        out_specs=[blk, blk, blk],
        out_shape=[jax.ShapeDtypeStruct((N_USER, HID), jnp.float32)] * 3,
    )(u, agg, sw, nw, b.reshape(1, -1), pw, pb.reshape(1, -1), fw, fb.reshape(1, -1))


def _final_body(hu_ref, aggf2_ref, ids_ref, p1parts_ref, p2parts_ref,
                news_ref, p1sw_ref, p1nw_ref, p1b_ref,
                f2sw_ref, f2nw_ref, f2b_ref,
                p2sw_ref, p2nw_ref, p2b_ref,
                lin1w_ref, lin1b_ref, lin2w_ref, lin2b_ref,
                clsw_ref, clsb_ref, out_ref, hg_acc, cnt_acc):
    i = pl.program_id(0)

    @pl.when(i == 0)
    def _():
        hg_acc[...] = jnp.zeros_like(hg_acc)
        cnt_acc[...] = jnp.zeros_like(cnt_acc)

    hu2 = (_dot_t(hu_ref[...], f2sw_ref[...])
           + _dot_t(aggf2_ref[...], f2nw_ref[...]) + f2b_ref[...])
    ids = ids_ref[0, 0, :]
    gid = jax.lax.broadcasted_iota(jnp.int32, (B, RB), 0)
    onehot = jnp.where(ids[None, :] == gid, 1.0, 0.0)
    hg_acc[...] += jnp.dot(onehot, hu2, preferred_element_type=jnp.float32)
    cnt_acc[...] += jnp.sum(onehot, axis=1, keepdims=True)

    @pl.when(i == NRB - 1)
    def _():
        aggp1 = jnp.max(p1parts_ref[...], axis=0)
        aggp2 = jnp.max(p2parts_ref[...], axis=0)
        news = news_ref[...]
        h_news = jnp.maximum(_dot_t(news, p1sw_ref[...])
                             + _dot_t(aggp1, p1nw_ref[...]) + p1b_ref[...], 0.0)
        h_news2 = (_dot_t(h_news, p2sw_ref[...])
                   + _dot_t(aggp2, p2nw_ref[...]) + p2b_ref[...])
        hg_in = h_news2 + hg_acc[...] / jnp.maximum(cnt_acc[...], 1.0)
        hg = jnp.maximum(_dot_t(hg_in, lin1w_ref[...]) + lin1b_ref[...], 0.0)
        nw = jnp.maximum(_dot_t(news, lin2w_ref[...]) + lin2b_ref[...], 0.0)
        z = jnp.concatenate([hg, nw], axis=1)
        logits = _dot_t(z, clsw_ref[...]) + clsb_ref[...]
        m = jnp.max(logits, axis=-1, keepdims=True)
        sh = logits - m
        lse = jnp.log(jnp.sum(jnp.exp(sh), axis=-1, keepdims=True))
        out_ref[...] = sh - lse


def _final_tc(hu, aggf2, ids, p1parts, p2parts, news,
              p1sw, p1nw, p1b, f2sw, f2nw, f2b, p2sw, p2nw, p2b,
              lin1w, lin1b, lin2w, lin2b, clsw, clsb):
    blk = pl.BlockSpec((RB, D), lambda i: (i, 0))
    full2 = pl.BlockSpec(None, lambda i: (0, 0))
    full3 = pl.BlockSpec(None, lambda i: (0, 0, 0))
    ids3 = ids.reshape(NRB, 1, RB)
    return pl.pallas_call(
        _final_body,
        grid=(NRB,),
        in_specs=[blk, blk, pl.BlockSpec((1, 1, RB), lambda i: (i, 0, 0)),
                  full3, full3, full2,
                  full2, full2, full2, full2, full2, full2,
                  full2, full2, full2, full2, full2, full2, full2,
                  full2, full2],
        out_specs=pl.BlockSpec(None, lambda i: (0, 0)),
        out_shape=jax.ShapeDtypeStruct((B, NCLS), jnp.float32),
        scratch_shapes=[pltpu.VMEM((B, HID), jnp.float32),
                        pltpu.VMEM((B, 1), jnp.float32)],
    )(hu, aggf2, ids3, p1parts, p2parts, news,
      p1sw, p1nw, p1b.reshape(1, -1), f2sw, f2nw, f2b.reshape(1, -1),
      p2sw, p2nw, p2b.reshape(1, -1), lin1w, lin1b.reshape(1, -1),
      lin2w, lin2b.reshape(1, -1), clsw, clsb.reshape(1, -1))


def kernel(news_feat, user_feat, p1_pool_w, p1_pool_b, p1_neigh_w, p1_self_w, p1_bias, f1_pool_w, f1_pool_b, f1_neigh_w, f1_self_w, f1_bias, p2_pool_w, p2_pool_b, p2_neigh_w, p2_self_w, p2_bias, f2_pool_w, f2_pool_b, f2_neigh_w, f2_self_w, f2_bias, lin1_w, lin1_b, lin2_w, lin2_b, cls_w, cls_b, posts_src, posts_dst, follows_src, follows_dst, user_graph_ids):
    npad = E_POSTS_PAD - E_POSTS
    psrc = jnp.concatenate([posts_src, jnp.zeros((npad,), jnp.int32)])
    pdst = jnp.concatenate([posts_dst, jnp.full((npad,), B, jnp.int32)])

    slab_s, slab_d, counts = _follows_prep(follows_src, follows_dst)

    hp1, hf1 = _pool2_tc(user_feat, p1_pool_w, p1_pool_b, f1_pool_w, f1_pool_b)

    p1parts = _posts_agg(hp1, psrc, pdst)
    aggf1 = _follows_agg(hf1, slab_s, slab_d, counts)

    h_user, hp2, hf2 = _huser_tc(user_feat, aggf1, f1_self_w, f1_neigh_w,
                                 f1_bias, p2_pool_w, p2_pool_b, f2_pool_w, f2_pool_b)

    p2parts = _posts_agg(hp2, psrc, pdst)
    aggf2 = _follows_agg(hf2, slab_s, slab_d, counts)

    return _final_tc(h_user, aggf2, user_graph_ids, p1parts, p2parts, news_feat,
                     p1_self_w, p1_neigh_w, p1_bias, f2_self_w, f2_neigh_w,
                     f2_bias, p2_self_w, p2_neigh_w, p2_bias,
                     lin1_w, lin1_b, lin2_w, lin2_b, cls_w, cls_b)
